# baseline (speedup 1.0000x reference)
import functools

import jax
import jax.numpy as jnp
from jax.experimental import pallas as pl
from jax.experimental.pallas import tpu as pltpu

_H = 768
_F = 1536
_E = 8
_TT = 256


def _router_body(x_ref, rw_ref, logits_ref, mask_ref):
    x = x_ref[...]
    logits = jax.lax.dot_general(
        x, rw_ref[...], (((1,), (0,)), ((), ())),
        precision=jax.lax.Precision.DEFAULT,
        preferred_element_type=jnp.float32,
    )
    logits_ref[...] = logits
    lane = jax.lax.broadcasted_iota(jnp.int32, logits.shape, 1)
    m1 = jnp.max(logits, axis=1, keepdims=True)
    idx1 = jnp.min(jnp.where(logits == m1, lane, _E), axis=1, keepdims=True)
    neg = jnp.full_like(logits, -jnp.inf)
    l2 = jnp.where(lane == idx1, neg, logits)
    m2 = jnp.max(l2, axis=1, keepdims=True)
    idx2 = jnp.min(jnp.where(l2 == m2, lane, _E), axis=1, keepdims=True)
    mask_ref[...] = ((lane == idx1) | (lane == idx2)).astype(jnp.float32)


def _expert_body(x_ref, upw_ref, upb_ref, dww_ref, dwb_ref, mask_ref,
                 out_ref, acc_ref):
    e = pl.program_id(0)
    t = pl.program_id(1)
    x = x_ref[...]
    up = jax.lax.dot_general(
        x, upw_ref[0], (((1,), (0,)), ((), ())),
        precision=jax.lax.Precision.HIGHEST,
        preferred_element_type=jnp.float32,
    )
    up = up + upb_ref[0]
    act = up * jax.nn.sigmoid(up)
    down = jax.lax.dot_general(
        act, dww_ref[0], (((1,), (0,)), ((), ())),
        precision=jax.lax.Precision.HIGHEST,
        preferred_element_type=jnp.float32,
    )
    down = down + dwb_ref[0]
    lane = jax.lax.broadcasted_iota(jnp.int32, mask_ref.shape, 1)
    m = jnp.sum(jnp.where(lane == e, mask_ref[...], 0.0), axis=1,
                keepdims=True)
    contrib = down * m
    sl = pl.ds(t * _TT, _TT)

    @pl.when(e == 0)
    def _():
        acc_ref[sl, :] = contrib

    @pl.when(e != 0)
    def _():
        acc_ref[sl, :] = acc_ref[sl, :] + contrib

    @pl.when(e == _E - 1)
    def _():
        out_ref[...] = acc_ref[sl, :]


@functools.partial(jax.jit, static_argnames=())
def kernel(hidden_states, router_w, up_w, up_b, down_w, down_b):
    b, s, d = hidden_states.shape
    t_total = b * s
    flat = hidden_states.reshape(t_total, d)

    logits, mask = pl.pallas_call(
        _router_body,
        out_shape=(
            jax.ShapeDtypeStruct((t_total, _E), jnp.float32),
            jax.ShapeDtypeStruct((t_total, _E), jnp.float32),
        ),
    )(flat, router_w)

    up_b3 = up_b.reshape(_E, 1, _F)
    down_b3 = down_b.reshape(_E, 1, _H)
    n_t = t_total // _TT

    final = pl.pallas_call(
        _expert_body,
        grid=(_E, n_t),
        in_specs=[
            pl.BlockSpec((_TT, _H), lambda e, t: (t, 0)),
            pl.BlockSpec((1, _H, _F), lambda e, t: (e, 0, 0)),
            pl.BlockSpec((1, 1, _F), lambda e, t: (e, 0, 0)),
            pl.BlockSpec((1, _F, _H), lambda e, t: (e, 0, 0)),
            pl.BlockSpec((1, 1, _H), lambda e, t: (e, 0, 0)),
            pl.BlockSpec((_TT, _E), lambda e, t: (t, 0)),
        ],
        out_specs=pl.BlockSpec((_TT, _H), lambda e, t: (t, 0)),
        out_shape=jax.ShapeDtypeStruct((t_total, _H), jnp.float32),
        scratch_shapes=[pltpu.VMEM((t_total, _H), jnp.float32)],
        compiler_params=pltpu.CompilerParams(
            dimension_semantics=("arbitrary", "arbitrary"),
        ),
    )(flat, up_w, up_b3, down_w, down_b3, mask)

    return final.reshape(b, s, d), logits


# speedup vs baseline: 3.9237x; 3.9237x over previous
import functools

import jax
import jax.numpy as jnp
from jax.experimental import pallas as pl
from jax.experimental.pallas import tpu as pltpu

_H = 768
_F = 1536
_E = 8
_TT = 256


def _router_body(x_ref, rw_ref, logits_ref, mask_ref):
    x = x_ref[...]
    logits = jax.lax.dot_general(
        x, rw_ref[...], (((1,), (0,)), ((), ())),
        precision=jax.lax.Precision.DEFAULT,
        preferred_element_type=jnp.float32,
    )
    logits_ref[...] = logits
    lane = jax.lax.broadcasted_iota(jnp.int32, logits.shape, 1)
    m1 = jnp.max(logits, axis=1, keepdims=True)
    idx1 = jnp.min(jnp.where(logits == m1, lane, _E), axis=1, keepdims=True)
    neg = jnp.full_like(logits, -jnp.inf)
    l2 = jnp.where(lane == idx1, neg, logits)
    m2 = jnp.max(l2, axis=1, keepdims=True)
    idx2 = jnp.min(jnp.where(l2 == m2, lane, _E), axis=1, keepdims=True)
    mask_ref[...] = ((lane == idx1) | (lane == idx2)).astype(jnp.float32)


def _expert_body(x_ref, upw_ref, upb_ref, dww_ref, dwb_ref, mask_ref,
                 out_ref, acc_ref):
    e = pl.program_id(0)
    t = pl.program_id(1)
    x = x_ref[...]
    up = jax.lax.dot_general(
        x, upw_ref[0], (((1,), (0,)), ((), ())),
        precision=jax.lax.Precision.DEFAULT,
        preferred_element_type=jnp.float32,
    )
    up = up + upb_ref[0]
    act = up * jax.nn.sigmoid(up)
    down = jax.lax.dot_general(
        act, dww_ref[0], (((1,), (0,)), ((), ())),
        precision=jax.lax.Precision.DEFAULT,
        preferred_element_type=jnp.float32,
    )
    down = down + dwb_ref[0]
    lane = jax.lax.broadcasted_iota(jnp.int32, mask_ref.shape, 1)
    m = jnp.sum(jnp.where(lane == e, mask_ref[...], 0.0), axis=1,
                keepdims=True)
    contrib = down * m
    sl = pl.ds(t * _TT, _TT)

    @pl.when(e == 0)
    def _():
        acc_ref[sl, :] = contrib

    @pl.when(e != 0)
    def _():
        acc_ref[sl, :] = acc_ref[sl, :] + contrib

    @pl.when(e == _E - 1)
    def _():
        out_ref[...] = acc_ref[sl, :]


@functools.partial(jax.jit, static_argnames=())
def kernel(hidden_states, router_w, up_w, up_b, down_w, down_b):
    b, s, d = hidden_states.shape
    t_total = b * s
    flat = hidden_states.reshape(t_total, d)

    logits, mask = pl.pallas_call(
        _router_body,
        out_shape=(
            jax.ShapeDtypeStruct((t_total, _E), jnp.float32),
            jax.ShapeDtypeStruct((t_total, _E), jnp.float32),
        ),
    )(flat, router_w)

    up_b3 = up_b.reshape(_E, 1, _F)
    down_b3 = down_b.reshape(_E, 1, _H)
    n_t = t_total // _TT

    final = pl.pallas_call(
        _expert_body,
        grid=(_E, n_t),
        in_specs=[
            pl.BlockSpec((_TT, _H), lambda e, t: (t, 0)),
            pl.BlockSpec((1, _H, _F), lambda e, t: (e, 0, 0)),
            pl.BlockSpec((1, 1, _F), lambda e, t: (e, 0, 0)),
            pl.BlockSpec((1, _F, _H), lambda e, t: (e, 0, 0)),
            pl.BlockSpec((1, 1, _H), lambda e, t: (e, 0, 0)),
            pl.BlockSpec((_TT, _E), lambda e, t: (t, 0)),
        ],
        out_specs=pl.BlockSpec((_TT, _H), lambda e, t: (t, 0)),
        out_shape=jax.ShapeDtypeStruct((t_total, _H), jnp.float32),
        scratch_shapes=[pltpu.VMEM((t_total, _H), jnp.float32)],
        compiler_params=pltpu.CompilerParams(
            dimension_semantics=("arbitrary", "arbitrary"),
        ),
    )(flat, up_w, up_b3, down_w, down_b3, mask)

    return final.reshape(b, s, d), logits
